# baseline (speedup 1.0000x reference)
import functools

import jax
import jax.numpy as jnp
from jax import lax
from jax.experimental import pallas as pl
from jax.experimental.pallas import tpu as pltpu
from jax.experimental.pallas import tpu_sc as plsc

K = 16
M1, M2 = 20, 10
NC, NS = 2, 16
NW = NC * NS
CB = 128
RC = 512


def _repack_body(v1_hbm, v2_hbm, v1f_hbm, v2f_hbm, buf2d, buf1d, sem,
                 *, n1, n2):
    wid = lax.axis_index("s") * NC + lax.axis_index("c")
    nc1, t1 = n1 // RC, n1 % RC
    nc2, t2 = n2 // RC, n2 % RC
    c1end = nc1 + (1 if t1 else 0)
    nchunks = c1end + nc2 + (1 if t2 else 0)

    def bridge(nrows):
        def row(n, _):
            buf1d[pl.ds(n * K, K)] = buf2d[n]
            return _
        lax.fori_loop(0, nrows, row, None)

    def move(src_hbm, dst_hbm, row0, nrows):
        pltpu.sync_copy(src_hbm.at[pl.ds(row0, nrows)],
                        buf2d.at[pl.ds(0, nrows)])
        bridge(nrows)
        pltpu.sync_copy(buf1d.at[pl.ds(0, nrows * K)],
                        dst_hbm.at[pl.ds(row0 * K, nrows * K)])

    def chunk(t, _):
        c = wid + NW * t

        @pl.when(c < nc1)
        def _():
            move(v1_hbm, v1f_hbm, c * RC, RC)
        if t1:
            @pl.when(c == nc1)
            def _():
                move(v1_hbm, v1f_hbm, nc1 * RC, t1)

        c2 = c - c1end

        @pl.when(jnp.logical_and(c2 >= 0, c2 < nc2))
        def _():
            move(v2_hbm, v2f_hbm, c2 * RC, RC)
        if t2:
            @pl.when(c2 == nc2)
            def _():
                move(v2_hbm, v2f_hbm, nc2 * RC, t2)
        return _

    lax.fori_loop(0, pl.cdiv(nchunks, NW), chunk, None)


def _fm_body(idx1_hbm, idx2_hbm, v1_2d, v2_2d, out_hbm,
             idx1_v, idx2_v, rows1_v, rows2_v, d_flat_v, out_v, sem1, sem2,
             *, batch):
    per_w = batch // NW
    nblk = per_w // CB
    wid = lax.axis_index("s") * NC + lax.axis_index("c")
    base = wid * per_w

    def block(t, _):
        eb = base + t * CB
        pltpu.sync_copy(idx1_hbm.at[pl.ds(eb * M1, CB * M1)], idx1_v)
        pltpu.sync_copy(idx2_hbm.at[pl.ds(eb * M2, CB * M2)], idx2_v)
        cp1 = pltpu.async_copy(v1_2d.at[idx1_v], rows1_v, sem1)
        cp2 = pltpu.async_copy(v2_2d.at[idx2_v], rows2_v, sem2)
        cp1.wait()
        cp2.wait()

        lanes = lax.iota(jnp.int32, K)
        zeros = jnp.zeros((K,), jnp.float32)

        def zero_pass(idx_v, rows_v, ngrp):
            def grp(g, _):
                ivec = idx_v[pl.ds(g * K, K)]
                mz = ivec == 0
                rowids = g * K + lanes
                for k in range(K):
                    plsc.store_scatter(
                        rows_v, [rowids, jnp.full((K,), k, jnp.int32)],
                        zeros, mask=mz)
                return _
            lax.fori_loop(0, ngrp, grp, None)

        zero_pass(idx1_v, rows1_v, CB * M1 // K)
        zero_pass(idx2_v, rows2_v, CB * M2 // K)

        def elem(i, _):
            s = jnp.zeros((K,), jnp.float32)
            q = jnp.zeros((K,), jnp.float32)
            for j in range(M1):
                r = rows1_v[i * M1 + j]
                s = s + r
                q = q + r * r
            for j in range(M2):
                r = rows2_v[i * M2 + j]
                s = s + r
                q = q + r * r
            d_flat_v[pl.ds(i * K, K)] = s * s - q
            return _

        lax.fori_loop(0, CB, elem, None)

        def red_grp(g, _):
            gbase = g * K * K
            acc = jnp.zeros((K,), jnp.float32)
            for k in range(K):
                col = plsc.load_gather(d_flat_v, [gbase + lanes * K + k])
                acc = acc + col
            out_v[pl.ds(g * K, K)] = 0.5 * acc
            return _

        lax.fori_loop(0, CB // K, red_grp, None)
        pltpu.sync_copy(out_v, out_hbm.at[pl.ds(eb, CB)])
        return _

    lax.fori_loop(0, nblk, block, None)


def kernel(idx1, idx2, V1, V2):
    batch = idx1.shape[0]
    n1, n2 = V1.shape[0], V2.shape[0]
    mesh = plsc.VectorSubcoreMesh(
        core_axis_name="c", subcore_axis_name="s",
        num_cores=NC, num_subcores=NS)

    repack = pl.kernel(
        functools.partial(_repack_body, n1=n1, n2=n2),
        out_type=(jax.ShapeDtypeStruct((n1 * K,), jnp.float32),
                  jax.ShapeDtypeStruct((n2 * K,), jnp.float32)),
        mesh=mesh,
        scratch_types=[
            pltpu.VMEM((RC, K), jnp.float32),
            pltpu.VMEM((RC * K,), jnp.float32),
            pltpu.SemaphoreType.DMA,
        ],
        compiler_params=pltpu.CompilerParams(
            needs_layout_passes=False, use_tc_tiling_on_sc=True),
    )
    v1f, v2f = repack(V1, V2)

    run = pl.kernel(
        functools.partial(_fm_body, batch=batch),
        out_type=jax.ShapeDtypeStruct((batch,), jnp.float32),
        mesh=mesh,
        scratch_types=[
            pltpu.VMEM((CB * M1,), jnp.int32),
            pltpu.VMEM((CB * M2,), jnp.int32),
            pltpu.VMEM((CB * M1, K), jnp.float32),
            pltpu.VMEM((CB * M2, K), jnp.float32),
            pltpu.VMEM((CB * K,), jnp.float32),
            pltpu.VMEM((CB,), jnp.float32),
            pltpu.SemaphoreType.DMA,
            pltpu.SemaphoreType.DMA,
        ],
        compiler_params=pltpu.CompilerParams(
            needs_layout_passes=False, use_tc_tiling_on_sc=False),
    )
    return run(idx1.reshape(-1), idx2.reshape(-1),
               v1f.reshape(n1, K), v2f.reshape(n2, K))


# speedup vs baseline: 1.0699x; 1.0699x over previous
import functools

import jax
import jax.numpy as jnp
from jax import lax
from jax.experimental import pallas as pl
from jax.experimental.pallas import tpu as pltpu
from jax.experimental.pallas import tpu_sc as plsc

K = 16
M1, M2 = 20, 10
NC, NS = 2, 16
NW = NC * NS
CB = 128
RC = 512


def _repack_body(v1_hbm, v2_hbm, v1f_hbm, v2f_hbm, buf2d, buf1d, sem,
                 *, n1, n2):
    wid = lax.axis_index("s") * NC + lax.axis_index("c")
    nc1, t1 = n1 // RC, n1 % RC
    nc2, t2 = n2 // RC, n2 % RC
    c1end = nc1 + (1 if t1 else 0)
    nchunks = c1end + nc2 + (1 if t2 else 0)

    def bridge(nrows):
        def row(n, _):
            buf1d[pl.ds(n * K, K)] = buf2d[n]
            return _
        lax.fori_loop(0, nrows, row, None, unroll=8)

    def move(src_hbm, dst_hbm, row0, nrows):
        pltpu.sync_copy(src_hbm.at[pl.ds(row0, nrows)],
                        buf2d.at[pl.ds(0, nrows)])
        bridge(nrows)
        pltpu.sync_copy(buf1d.at[pl.ds(0, nrows * K)],
                        dst_hbm.at[pl.ds(row0 * K, nrows * K)])

    def chunk(t, _):
        c = wid + NW * t

        @pl.when(c < nc1)
        def _():
            move(v1_hbm, v1f_hbm, c * RC, RC)
        if t1:
            @pl.when(c == nc1)
            def _():
                move(v1_hbm, v1f_hbm, nc1 * RC, t1)

        c2 = c - c1end

        @pl.when(jnp.logical_and(c2 >= 0, c2 < nc2))
        def _():
            move(v2_hbm, v2f_hbm, c2 * RC, RC)
        if t2:
            @pl.when(c2 == nc2)
            def _():
                move(v2_hbm, v2f_hbm, nc2 * RC, t2)
        return _

    lax.fori_loop(0, pl.cdiv(nchunks, NW), chunk, None)


def _fm_body(idx1_hbm, idx2_hbm, v1_2d, v2_2d, out_hbm,
             idx1_v, idx2_v, rows1_v, rows2_v, d_flat_v, out_v, sem1, sem2,
             *, batch):
    per_w = batch // NW
    nblk = per_w // CB
    wid = lax.axis_index("s") * NC + lax.axis_index("c")
    base = wid * per_w

    def block(t, _):
        eb = base + t * CB
        pltpu.sync_copy(idx1_hbm.at[pl.ds(eb * M1, CB * M1)], idx1_v)
        pltpu.sync_copy(idx2_hbm.at[pl.ds(eb * M2, CB * M2)], idx2_v)
        cp1 = pltpu.async_copy(v1_2d.at[idx1_v], rows1_v, sem1)
        cp2 = pltpu.async_copy(v2_2d.at[idx2_v], rows2_v, sem2)
        cp1.wait()
        cp2.wait()

        lanes = lax.iota(jnp.int32, K)
        zeros = jnp.zeros((K,), jnp.float32)

        def zero_pass(idx_v, rows_v, ngrp):
            def grp(g, _):
                ivec = idx_v[pl.ds(g * K, K)]
                mz = ivec == 0
                rowids = g * K + lanes
                for k in range(K):
                    plsc.store_scatter(
                        rows_v, [rowids, jnp.full((K,), k, jnp.int32)],
                        zeros, mask=mz)
                return _
            lax.fori_loop(0, ngrp, grp, None)

        zero_pass(idx1_v, rows1_v, CB * M1 // K)
        zero_pass(idx2_v, rows2_v, CB * M2 // K)

        def elem(i, _):
            s = jnp.zeros((K,), jnp.float32)
            q = jnp.zeros((K,), jnp.float32)
            for j in range(M1):
                r = rows1_v[i * M1 + j]
                s = s + r
                q = q + r * r
            for j in range(M2):
                r = rows2_v[i * M2 + j]
                s = s + r
                q = q + r * r
            d_flat_v[pl.ds(i * K, K)] = s * s - q
            return _

        lax.fori_loop(0, CB, elem, None)

        def red_grp(g, _):
            gbase = g * K * K
            acc = jnp.zeros((K,), jnp.float32)
            for k in range(K):
                col = plsc.load_gather(d_flat_v, [gbase + lanes * K + k])
                acc = acc + col
            out_v[pl.ds(g * K, K)] = 0.5 * acc
            return _

        lax.fori_loop(0, CB // K, red_grp, None)
        pltpu.sync_copy(out_v, out_hbm.at[pl.ds(eb, CB)])
        return _

    lax.fori_loop(0, nblk, block, None)


def kernel(idx1, idx2, V1, V2):
    batch = idx1.shape[0]
    n1, n2 = V1.shape[0], V2.shape[0]
    mesh = plsc.VectorSubcoreMesh(
        core_axis_name="c", subcore_axis_name="s",
        num_cores=NC, num_subcores=NS)

    repack = pl.kernel(
        functools.partial(_repack_body, n1=n1, n2=n2),
        out_type=(jax.ShapeDtypeStruct((n1 * K,), jnp.float32),
                  jax.ShapeDtypeStruct((n2 * K,), jnp.float32)),
        mesh=mesh,
        scratch_types=[
            pltpu.VMEM((RC, K), jnp.float32),
            pltpu.VMEM((RC * K,), jnp.float32),
            pltpu.SemaphoreType.DMA,
        ],
        compiler_params=pltpu.CompilerParams(
            needs_layout_passes=False, use_tc_tiling_on_sc=True),
    )
    v1f, v2f = repack(V1, V2)

    run = pl.kernel(
        functools.partial(_fm_body, batch=batch),
        out_type=jax.ShapeDtypeStruct((batch,), jnp.float32),
        mesh=mesh,
        scratch_types=[
            pltpu.VMEM((CB * M1,), jnp.int32),
            pltpu.VMEM((CB * M2,), jnp.int32),
            pltpu.VMEM((CB * M1, K), jnp.float32),
            pltpu.VMEM((CB * M2, K), jnp.float32),
            pltpu.VMEM((CB * K,), jnp.float32),
            pltpu.VMEM((CB,), jnp.float32),
            pltpu.SemaphoreType.DMA,
            pltpu.SemaphoreType.DMA,
        ],
        compiler_params=pltpu.CompilerParams(
            needs_layout_passes=False, use_tc_tiling_on_sc=False),
    )
    return run(idx1.reshape(-1), idx2.reshape(-1),
               v1f.reshape(n1, K), v2f.reshape(n2, K))


# speedup vs baseline: 1.2243x; 1.1444x over previous
import functools

import jax
import jax.numpy as jnp
from jax import lax
from jax.experimental import pallas as pl
from jax.experimental.pallas import tpu as pltpu
from jax.experimental.pallas import tpu_sc as plsc

K = 16
M1, M2 = 20, 10
NC, NS = 2, 16
NW = NC * NS
CB = 128
RC = 256


def _repack_body(v1_hbm, v2_hbm, v1f_hbm, v2f_hbm,
                 buf2d_a, buf2d_b, buf1d_a, buf1d_b,
                 semr_a, semr_b, semw_a, semw_b,
                 *, n1, n2):
    wid = lax.axis_index("s") * NC + lax.axis_index("c")
    nc1, t1 = n1 // RC, n1 % RC
    nc2, t2 = n2 // RC, n2 % RC
    nfull = nc1 + nc2
    buf2d = (buf2d_a, buf2d_b)
    buf1d = (buf1d_a, buf1d_b)
    semr = (semr_a, semr_b)
    semw = (semw_a, semw_b)

    def rd_desc(c, ph):
        def mk(src_hbm, row0):
            return pltpu.make_async_copy(
                src_hbm.at[pl.ds(row0, RC)], buf2d[ph], semr[ph])
        return mk

    def start_read(c, ph):
        @pl.when(c < nc1)
        def _():
            rd_desc(c, ph)(v1_hbm, c * RC).start()
        @pl.when(jnp.logical_and(c >= nc1, c < nfull))
        def _():
            rd_desc(c, ph)(v2_hbm, (c - nc1) * RC).start()

    def wait_read(c, ph):
        pltpu.make_async_copy(v1_hbm.at[pl.ds(0, RC)], buf2d[ph],
                              semr[ph]).wait()

    def start_write(c, ph):
        @pl.when(c < nc1)
        def _():
            pltpu.make_async_copy(
                buf1d[ph], v1f_hbm.at[pl.ds(c * RC * K, RC * K)],
                semw[ph]).start()
        @pl.when(jnp.logical_and(c >= nc1, c < nfull))
        def _():
            pltpu.make_async_copy(
                buf1d[ph], v2f_hbm.at[pl.ds((c - nc1) * RC * K, RC * K)],
                semw[ph]).start()

    def wait_write(ph):
        pltpu.make_async_copy(buf1d[ph], v1f_hbm.at[pl.ds(0, RC * K)],
                              semw[ph]).wait()

    def bridge(src2d, dst1d, nrows):
        def row(n, _):
            dst1d[pl.ds(n * K, K)] = src2d[n]
            return _
        lax.fori_loop(0, nrows, row, None, unroll=8)

    my_n = (nfull - wid + NW - 1) // NW
    tmax = (nfull + NW - 1) // NW
    start_read(wid, 0)

    def pair(p, _):
        for ph in range(2):
            t = 2 * p + ph
            c = wid + NW * t
            cn = c + NW

            @pl.when(c < nfull)
            def _():
                wait_read(c, ph)

            @pl.when(cn < nfull)
            def _():
                start_read(cn, 1 - ph)

            @pl.when(jnp.logical_and(t >= 2, c < nfull))
            def _():
                wait_write(ph)

            @pl.when(c < nfull)
            def _():
                bridge(buf2d[ph], buf1d[ph], RC)
                start_write(c, ph)
        return _

    lax.fori_loop(0, (tmax + 1) // 2, pair, None)

    for ph in range(2):
        @pl.when(my_n > ph)
        def _():
            wait_write(ph)

    if t1:
        @pl.when(wid == 0)
        def _():
            pltpu.sync_copy(v1_hbm.at[pl.ds(nc1 * RC, t1)],
                            buf2d_a.at[pl.ds(0, t1)])
            bridge(buf2d_a, buf1d_a, t1)
            pltpu.sync_copy(buf1d_a.at[pl.ds(0, t1 * K)],
                            v1f_hbm.at[pl.ds(nc1 * RC * K, t1 * K)])
    if t2:
        @pl.when(wid == 1)
        def _():
            pltpu.sync_copy(v2_hbm.at[pl.ds(nc2 * RC, t2)],
                            buf2d_a.at[pl.ds(0, t2)])
            bridge(buf2d_a, buf1d_a, t2)
            pltpu.sync_copy(buf1d_a.at[pl.ds(0, t2 * K)],
                            v2f_hbm.at[pl.ds(nc2 * RC * K, t2 * K)])


def _fm_body(idx1_hbm, idx2_hbm, v1_2d, v2_2d, out_hbm,
             idx1_v, idx2_v, rows1_v, rows2_v, d_flat_v, out_v, sem1, sem2,
             *, batch):
    per_w = batch // NW
    nblk = per_w // CB
    wid = lax.axis_index("s") * NC + lax.axis_index("c")
    base = wid * per_w

    def block(t, _):
        eb = base + t * CB
        pltpu.sync_copy(idx1_hbm.at[pl.ds(eb * M1, CB * M1)], idx1_v)
        pltpu.sync_copy(idx2_hbm.at[pl.ds(eb * M2, CB * M2)], idx2_v)
        cp1 = pltpu.async_copy(v1_2d.at[idx1_v], rows1_v, sem1)
        cp2 = pltpu.async_copy(v2_2d.at[idx2_v], rows2_v, sem2)
        cp1.wait()
        cp2.wait()

        lanes = lax.iota(jnp.int32, K)
        zeros = jnp.zeros((K,), jnp.float32)

        def zero_pass(idx_v, rows_v, ngrp):
            def grp(g, _):
                ivec = idx_v[pl.ds(g * K, K)]
                mz = ivec == 0
                rowids = g * K + lanes
                for k in range(K):
                    plsc.store_scatter(
                        rows_v, [rowids, jnp.full((K,), k, jnp.int32)],
                        zeros, mask=mz)
                return _
            lax.fori_loop(0, ngrp, grp, None)

        zero_pass(idx1_v, rows1_v, CB * M1 // K)
        zero_pass(idx2_v, rows2_v, CB * M2 // K)

        def elem(i, _):
            s = jnp.zeros((K,), jnp.float32)
            q = jnp.zeros((K,), jnp.float32)
            for j in range(M1):
                r = rows1_v[i * M1 + j]
                s = s + r
                q = q + r * r
            for j in range(M2):
                r = rows2_v[i * M2 + j]
                s = s + r
                q = q + r * r
            d_flat_v[pl.ds(i * K, K)] = s * s - q
            return _

        lax.fori_loop(0, CB, elem, None)

        def red_grp(g, _):
            gbase = g * K * K
            acc = jnp.zeros((K,), jnp.float32)
            for k in range(K):
                col = plsc.load_gather(d_flat_v, [gbase + lanes * K + k])
                acc = acc + col
            out_v[pl.ds(g * K, K)] = 0.5 * acc
            return _

        lax.fori_loop(0, CB // K, red_grp, None)
        pltpu.sync_copy(out_v, out_hbm.at[pl.ds(eb, CB)])
        return _

    lax.fori_loop(0, nblk, block, None)


def kernel(idx1, idx2, V1, V2):
    batch = idx1.shape[0]
    n1, n2 = V1.shape[0], V2.shape[0]
    mesh = plsc.VectorSubcoreMesh(
        core_axis_name="c", subcore_axis_name="s",
        num_cores=NC, num_subcores=NS)

    repack = pl.kernel(
        functools.partial(_repack_body, n1=n1, n2=n2),
        out_type=(jax.ShapeDtypeStruct((n1 * K,), jnp.float32),
                  jax.ShapeDtypeStruct((n2 * K,), jnp.float32)),
        mesh=mesh,
        scratch_types=[
            pltpu.VMEM((RC, K), jnp.float32),
            pltpu.VMEM((RC, K), jnp.float32),
            pltpu.VMEM((RC * K,), jnp.float32),
            pltpu.VMEM((RC * K,), jnp.float32),
            pltpu.SemaphoreType.DMA,
            pltpu.SemaphoreType.DMA,
            pltpu.SemaphoreType.DMA,
            pltpu.SemaphoreType.DMA,
        ],
        compiler_params=pltpu.CompilerParams(
            needs_layout_passes=False, use_tc_tiling_on_sc=True),
    )
    v1f, v2f = repack(V1, V2)

    run = pl.kernel(
        functools.partial(_fm_body, batch=batch),
        out_type=jax.ShapeDtypeStruct((batch,), jnp.float32),
        mesh=mesh,
        scratch_types=[
            pltpu.VMEM((CB * M1,), jnp.int32),
            pltpu.VMEM((CB * M2,), jnp.int32),
            pltpu.VMEM((CB * M1, K), jnp.float32),
            pltpu.VMEM((CB * M2, K), jnp.float32),
            pltpu.VMEM((CB * K,), jnp.float32),
            pltpu.VMEM((CB,), jnp.float32),
            pltpu.SemaphoreType.DMA,
            pltpu.SemaphoreType.DMA,
        ],
        compiler_params=pltpu.CompilerParams(
            needs_layout_passes=False, use_tc_tiling_on_sc=False),
    )
    return run(idx1.reshape(-1), idx2.reshape(-1),
               v1f.reshape(n1, K), v2f.reshape(n2, K))


# speedup vs baseline: 1.2412x; 1.0138x over previous
import functools

import jax
import jax.numpy as jnp
from jax import lax
from jax.experimental import pallas as pl
from jax.experimental.pallas import tpu as pltpu
from jax.experimental.pallas import tpu_sc as plsc

K = 16
M1, M2 = 20, 10
NC, NS = 2, 16
NW = NC * NS
CB = 64
RC = 256


def _repack_body(v1_hbm, v2_hbm, v1f_hbm, v2f_hbm,
                 buf2d_a, buf2d_b, buf1d_a, buf1d_b,
                 semr_a, semr_b, semw_a, semw_b,
                 *, n1, n2):
    wid = lax.axis_index("s") * NC + lax.axis_index("c")
    nc1, t1 = n1 // RC, n1 % RC
    nc2, t2 = n2 // RC, n2 % RC
    nfull = nc1 + nc2
    buf2d = (buf2d_a, buf2d_b)
    buf1d = (buf1d_a, buf1d_b)
    semr = (semr_a, semr_b)
    semw = (semw_a, semw_b)

    def rd_desc(c, ph):
        def mk(src_hbm, row0):
            return pltpu.make_async_copy(
                src_hbm.at[pl.ds(row0, RC)], buf2d[ph], semr[ph])
        return mk

    def start_read(c, ph):
        @pl.when(c < nc1)
        def _():
            rd_desc(c, ph)(v1_hbm, c * RC).start()
        @pl.when(jnp.logical_and(c >= nc1, c < nfull))
        def _():
            rd_desc(c, ph)(v2_hbm, (c - nc1) * RC).start()

    def wait_read(c, ph):
        pltpu.make_async_copy(v1_hbm.at[pl.ds(0, RC)], buf2d[ph],
                              semr[ph]).wait()

    def start_write(c, ph):
        @pl.when(c < nc1)
        def _():
            pltpu.make_async_copy(
                buf1d[ph], v1f_hbm.at[pl.ds(c * RC * K, RC * K)],
                semw[ph]).start()
        @pl.when(jnp.logical_and(c >= nc1, c < nfull))
        def _():
            pltpu.make_async_copy(
                buf1d[ph], v2f_hbm.at[pl.ds((c - nc1) * RC * K, RC * K)],
                semw[ph]).start()

    def wait_write(ph):
        pltpu.make_async_copy(buf1d[ph], v1f_hbm.at[pl.ds(0, RC * K)],
                              semw[ph]).wait()

    def bridge(src2d, dst1d, nrows):
        def row(n, _):
            dst1d[pl.ds(n * K, K)] = src2d[n]
            return _
        lax.fori_loop(0, nrows, row, None, unroll=8)

    my_n = (nfull - wid + NW - 1) // NW
    tmax = (nfull + NW - 1) // NW
    start_read(wid, 0)

    def pair(p, _):
        for ph in range(2):
            t = 2 * p + ph
            c = wid + NW * t
            cn = c + NW

            @pl.when(c < nfull)
            def _():
                wait_read(c, ph)

            @pl.when(cn < nfull)
            def _():
                start_read(cn, 1 - ph)

            @pl.when(jnp.logical_and(t >= 2, c < nfull))
            def _():
                wait_write(ph)

            @pl.when(c < nfull)
            def _():
                bridge(buf2d[ph], buf1d[ph], RC)
                start_write(c, ph)
        return _

    lax.fori_loop(0, (tmax + 1) // 2, pair, None)

    for ph in range(2):
        @pl.when(my_n > ph)
        def _():
            wait_write(ph)

    if t1:
        @pl.when(wid == 0)
        def _():
            pltpu.sync_copy(v1_hbm.at[pl.ds(nc1 * RC, t1)],
                            buf2d_a.at[pl.ds(0, t1)])
            bridge(buf2d_a, buf1d_a, t1)
            pltpu.sync_copy(buf1d_a.at[pl.ds(0, t1 * K)],
                            v1f_hbm.at[pl.ds(nc1 * RC * K, t1 * K)])
    if t2:
        @pl.when(wid == 1)
        def _():
            pltpu.sync_copy(v2_hbm.at[pl.ds(nc2 * RC, t2)],
                            buf2d_a.at[pl.ds(0, t2)])
            bridge(buf2d_a, buf1d_a, t2)
            pltpu.sync_copy(buf1d_a.at[pl.ds(0, t2 * K)],
                            v2f_hbm.at[pl.ds(nc2 * RC * K, t2 * K)])


def _fm_body(idx1_hbm, idx2_hbm, v1_2d, v2_2d, out_hbm,
             idx1_a, idx1_b, idx2_a, idx2_b,
             rows1_a, rows1_b, rows2_a, rows2_b,
             d_flat_v, out_v,
             sem1_a, sem1_b, sem2_a, sem2_b,
             *, batch):
    per_w = batch // NW
    nblk = per_w // CB
    wid = lax.axis_index("s") * NC + lax.axis_index("c")
    base = wid * per_w
    idx1_v = (idx1_a, idx1_b)
    idx2_v = (idx2_a, idx2_b)
    rows1_v = (rows1_a, rows1_b)
    rows2_v = (rows2_a, rows2_b)
    sem1 = (sem1_a, sem1_b)
    sem2 = (sem2_a, sem2_b)

    def start_gathers(t, ph):
        eb = base + t * CB
        pltpu.sync_copy(idx1_hbm.at[pl.ds(eb * M1, CB * M1)], idx1_v[ph])
        pltpu.sync_copy(idx2_hbm.at[pl.ds(eb * M2, CB * M2)], idx2_v[ph])
        pltpu.make_async_copy(v1_2d.at[idx1_v[ph]], rows1_v[ph],
                              sem1[ph]).start()
        pltpu.make_async_copy(v2_2d.at[idx2_v[ph]], rows2_v[ph],
                              sem2[ph]).start()

    def wait_gathers(ph):
        pltpu.make_async_copy(v1_2d.at[idx1_v[ph]], rows1_v[ph],
                              sem1[ph]).wait()
        pltpu.make_async_copy(v2_2d.at[idx2_v[ph]], rows2_v[ph],
                              sem2[ph]).wait()

    lanes = lax.iota(jnp.int32, K)
    zeros = jnp.zeros((K,), jnp.float32)

    def zero_pass(idx_v, rows_v, ngrp):
        def grp(g, _):
            ivec = idx_v[pl.ds(g * K, K)]
            mz = ivec == 0
            rowids = g * K + lanes
            for k in range(K):
                plsc.store_scatter(
                    rows_v, [rowids, jnp.full((K,), k, jnp.int32)],
                    zeros, mask=mz)
            return _
        lax.fori_loop(0, ngrp, grp, None, unroll=2)

    def compute(t, ph):
        zero_pass(idx1_v[ph], rows1_v[ph], CB * M1 // K)
        zero_pass(idx2_v[ph], rows2_v[ph], CB * M2 // K)
        r1 = rows1_v[ph]
        r2 = rows2_v[ph]

        def elem(i, _):
            s = jnp.zeros((K,), jnp.float32)
            q = jnp.zeros((K,), jnp.float32)
            for j in range(M1):
                r = r1[i * M1 + j]
                s = s + r
                q = q + r * r
            for j in range(M2):
                r = r2[i * M2 + j]
                s = s + r
                q = q + r * r
            d_flat_v[pl.ds(i * K, K)] = s * s - q
            return _

        lax.fori_loop(0, CB, elem, None, unroll=2)

        def red_grp(g, _):
            gbase = g * K * K
            acc = jnp.zeros((K,), jnp.float32)
            for k in range(K):
                col = plsc.load_gather(d_flat_v, [gbase + lanes * K + k])
                acc = acc + col
            out_v[pl.ds(t * CB + g * K, K)] = 0.5 * acc
            return _

        lax.fori_loop(0, CB // K, red_grp, None)

    start_gathers(0, 0)

    def pair(p, _):
        for ph in range(2):
            t = 2 * p + ph

            @pl.when(t < nblk)
            def _():
                wait_gathers(ph)

                @pl.when(t + 1 < nblk)
                def _():
                    start_gathers(t + 1, 1 - ph)

                compute(t, ph)
        return _

    lax.fori_loop(0, (nblk + 1) // 2, pair, None)
    pltpu.sync_copy(out_v, out_hbm.at[pl.ds(base, per_w)])


def kernel(idx1, idx2, V1, V2):
    batch = idx1.shape[0]
    n1, n2 = V1.shape[0], V2.shape[0]
    mesh = plsc.VectorSubcoreMesh(
        core_axis_name="c", subcore_axis_name="s",
        num_cores=NC, num_subcores=NS)

    repack = pl.kernel(
        functools.partial(_repack_body, n1=n1, n2=n2),
        out_type=(jax.ShapeDtypeStruct((n1 * K,), jnp.float32),
                  jax.ShapeDtypeStruct((n2 * K,), jnp.float32)),
        mesh=mesh,
        scratch_types=[
            pltpu.VMEM((RC, K), jnp.float32),
            pltpu.VMEM((RC, K), jnp.float32),
            pltpu.VMEM((RC * K,), jnp.float32),
            pltpu.VMEM((RC * K,), jnp.float32),
            pltpu.SemaphoreType.DMA,
            pltpu.SemaphoreType.DMA,
            pltpu.SemaphoreType.DMA,
            pltpu.SemaphoreType.DMA,
        ],
        compiler_params=pltpu.CompilerParams(
            needs_layout_passes=False, use_tc_tiling_on_sc=True),
    )
    v1f, v2f = repack(V1, V2)

    run = pl.kernel(
        functools.partial(_fm_body, batch=batch),
        out_type=jax.ShapeDtypeStruct((batch,), jnp.float32),
        mesh=mesh,
        scratch_types=[
            pltpu.VMEM((CB * M1,), jnp.int32),
            pltpu.VMEM((CB * M1,), jnp.int32),
            pltpu.VMEM((CB * M2,), jnp.int32),
            pltpu.VMEM((CB * M2,), jnp.int32),
            pltpu.VMEM((CB * M1, K), jnp.float32),
            pltpu.VMEM((CB * M1, K), jnp.float32),
            pltpu.VMEM((CB * M2, K), jnp.float32),
            pltpu.VMEM((CB * M2, K), jnp.float32),
            pltpu.VMEM((CB * K,), jnp.float32),
            pltpu.VMEM((batch // NW,), jnp.float32),
            pltpu.SemaphoreType.DMA,
            pltpu.SemaphoreType.DMA,
            pltpu.SemaphoreType.DMA,
            pltpu.SemaphoreType.DMA,
        ],
        compiler_params=pltpu.CompilerParams(
            needs_layout_passes=False, use_tc_tiling_on_sc=False),
    )
    return run(idx1.reshape(-1), idx2.reshape(-1),
               v1f.reshape(n1, K), v2f.reshape(n2, K))


# speedup vs baseline: 1.4814x; 1.1935x over previous
import functools

import jax
import jax.numpy as jnp
from jax import lax
from jax.experimental import pallas as pl
from jax.experimental.pallas import tpu as pltpu
from jax.experimental.pallas import tpu_sc as plsc

K = 16
M1, M2 = 20, 10
NC, NS = 2, 16
NW = NC * NS
CB = 64
RC = 256


def _repack_body(v1_hbm, v2_hbm, v1f_hbm, v2f_hbm,
                 buf2d_a, buf2d_b, buf1d_a, buf1d_b,
                 semr_a, semr_b, semw_a, semw_b,
                 *, n1, n2):
    wid = lax.axis_index("s") * NC + lax.axis_index("c")
    nc1, t1 = n1 // RC, n1 % RC
    nc2, t2 = n2 // RC, n2 % RC
    nfull = nc1 + nc2
    buf2d = (buf2d_a, buf2d_b)
    buf1d = (buf1d_a, buf1d_b)
    semr = (semr_a, semr_b)
    semw = (semw_a, semw_b)

    def rd_desc(c, ph):
        def mk(src_hbm, row0):
            return pltpu.make_async_copy(
                src_hbm.at[pl.ds(row0, RC)], buf2d[ph], semr[ph])
        return mk

    def start_read(c, ph):
        @pl.when(c < nc1)
        def _():
            rd_desc(c, ph)(v1_hbm, c * RC).start()
        @pl.when(jnp.logical_and(c >= nc1, c < nfull))
        def _():
            rd_desc(c, ph)(v2_hbm, (c - nc1) * RC).start()

    def wait_read(c, ph):
        pltpu.make_async_copy(v1_hbm.at[pl.ds(0, RC)], buf2d[ph],
                              semr[ph]).wait()

    def start_write(c, ph):
        @pl.when(c < nc1)
        def _():
            pltpu.make_async_copy(
                buf1d[ph], v1f_hbm.at[pl.ds(c * RC * K, RC * K)],
                semw[ph]).start()
        @pl.when(jnp.logical_and(c >= nc1, c < nfull))
        def _():
            pltpu.make_async_copy(
                buf1d[ph], v2f_hbm.at[pl.ds((c - nc1) * RC * K, RC * K)],
                semw[ph]).start()

    def wait_write(ph):
        pltpu.make_async_copy(buf1d[ph], v1f_hbm.at[pl.ds(0, RC * K)],
                              semw[ph]).wait()

    def bridge(src2d, dst1d, nrows):
        def row(n, _):
            dst1d[pl.ds(n * K, K)] = src2d[n]
            return _
        lax.fori_loop(0, nrows, row, None, unroll=8)

    my_n = (nfull - wid + NW - 1) // NW
    tmax = (nfull + NW - 1) // NW
    start_read(wid, 0)

    def pair(p, _):
        for ph in range(2):
            t = 2 * p + ph
            c = wid + NW * t
            cn = c + NW

            @pl.when(c < nfull)
            def _():
                wait_read(c, ph)

            @pl.when(cn < nfull)
            def _():
                start_read(cn, 1 - ph)

            @pl.when(jnp.logical_and(t >= 2, c < nfull))
            def _():
                wait_write(ph)

            @pl.when(c < nfull)
            def _():
                bridge(buf2d[ph], buf1d[ph], RC)
                start_write(c, ph)
        return _

    lax.fori_loop(0, (tmax + 1) // 2, pair, None)

    for ph in range(2):
        @pl.when(my_n > ph)
        def _():
            wait_write(ph)

    if t1:
        @pl.when(wid == 0)
        def _():
            pltpu.sync_copy(v1_hbm.at[pl.ds(nc1 * RC, t1)],
                            buf2d_a.at[pl.ds(0, t1)])
            bridge(buf2d_a, buf1d_a, t1)
            pltpu.sync_copy(buf1d_a.at[pl.ds(0, t1 * K)],
                            v1f_hbm.at[pl.ds(nc1 * RC * K, t1 * K)])
    if t2:
        @pl.when(wid == 1)
        def _():
            pltpu.sync_copy(v2_hbm.at[pl.ds(nc2 * RC, t2)],
                            buf2d_a.at[pl.ds(0, t2)])
            bridge(buf2d_a, buf1d_a, t2)
            pltpu.sync_copy(buf1d_a.at[pl.ds(0, t2 * K)],
                            v2f_hbm.at[pl.ds(nc2 * RC * K, t2 * K)])


def _fm_body(idx1_hbm, idx2_hbm, v1_2d, v2_2d, out_hbm,
             idx1_a, idx1_b, idx2_a, idx2_b,
             rows1_a, rows1_b, rows2_a, rows2_b,
             d_flat_v, out_v,
             sem1_a, sem1_b, sem2_a, sem2_b,
             *, batch):
    per_w = batch // NW
    nblk = per_w // CB
    wid = lax.axis_index("s") * NC + lax.axis_index("c")
    base = wid * per_w
    idx1_v = (idx1_a, idx1_b)
    idx2_v = (idx2_a, idx2_b)
    rows1_v = (rows1_a, rows1_b)
    rows2_v = (rows2_a, rows2_b)
    sem1 = (sem1_a, sem1_b)
    sem2 = (sem2_a, sem2_b)

    def start_gathers(t, ph):
        eb = base + t * CB
        pltpu.sync_copy(idx1_hbm.at[pl.ds(eb * M1, CB * M1)], idx1_v[ph])
        pltpu.sync_copy(idx2_hbm.at[pl.ds(eb * M2, CB * M2)], idx2_v[ph])
        pltpu.make_async_copy(v1_2d.at[idx1_v[ph]], rows1_v[ph],
                              sem1[ph]).start()
        pltpu.make_async_copy(v2_2d.at[idx2_v[ph]], rows2_v[ph],
                              sem2[ph]).start()

    def wait_gathers(ph):
        pltpu.make_async_copy(v1_2d.at[idx1_v[ph]], rows1_v[ph],
                              sem1[ph]).wait()
        pltpu.make_async_copy(v2_2d.at[idx2_v[ph]], rows2_v[ph],
                              sem2[ph]).wait()

    lanes = lax.iota(jnp.int32, K)
    zeros = jnp.zeros((K,), jnp.float32)

    def zero_pass(idx_v, rows_v, ngrp):
        def grp(g, _):
            ivec = idx_v[pl.ds(g * K, K)]
            mz = ivec == 0
            rowids = g * K + lanes
            for k in range(K):
                plsc.store_scatter(
                    rows_v, [rowids, jnp.full((K,), k, jnp.int32)],
                    zeros, mask=mz)
            return _
        lax.fori_loop(0, ngrp, grp, None, unroll=2)

    def compute(t, ph):
        zero_pass(idx1_v[ph], rows1_v[ph], CB * M1 // K)
        zero_pass(idx2_v[ph], rows2_v[ph], CB * M2 // K)
        r1 = rows1_v[ph]
        r2 = rows2_v[ph]

        def elem(i, _):
            s = jnp.zeros((K,), jnp.float32)
            q = jnp.zeros((K,), jnp.float32)
            for j in range(M1):
                r = r1[i * M1 + j]
                s = s + r
                q = q + r * r
            for j in range(M2):
                r = r2[i * M2 + j]
                s = s + r
                q = q + r * r
            d_flat_v[pl.ds(i * K, K)] = s * s - q
            return _

        lax.fori_loop(0, CB, elem, None, unroll=2)

        def red_grp(g, _):
            gbase = g * K * K
            acc = jnp.zeros((K,), jnp.float32)
            for k in range(K):
                col = plsc.load_gather(d_flat_v, [gbase + lanes * K + k])
                acc = acc + col
            out_v[pl.ds(t * CB + g * K, K)] = 0.5 * acc
            return _

        lax.fori_loop(0, CB // K, red_grp, None)

    start_gathers(0, 0)

    def pair(p, _):
        for ph in range(2):
            t = 2 * p + ph

            @pl.when(t < nblk)
            def _():
                wait_gathers(ph)

                @pl.when(t + 1 < nblk)
                def _():
                    start_gathers(t + 1, 1 - ph)

                compute(t, ph)
        return _

    lax.fori_loop(0, (nblk + 1) // 2, pair, None)
    pltpu.sync_copy(out_v, out_hbm.at[pl.ds(base, per_w)])


def kernel(idx1, idx2, V1, V2):
    batch = idx1.shape[0]
    n1, n2 = V1.shape[0], V2.shape[0]
    mesh = plsc.VectorSubcoreMesh(
        core_axis_name="c", subcore_axis_name="s",
        num_cores=NC, num_subcores=NS)

    run = pl.kernel(
        functools.partial(_fm_body, batch=batch),
        out_type=jax.ShapeDtypeStruct((batch,), jnp.float32),
        mesh=mesh,
        scratch_types=[
            pltpu.VMEM((CB * M1,), jnp.int32),
            pltpu.VMEM((CB * M1,), jnp.int32),
            pltpu.VMEM((CB * M2,), jnp.int32),
            pltpu.VMEM((CB * M2,), jnp.int32),
            pltpu.VMEM((CB * M1, K), jnp.float32),
            pltpu.VMEM((CB * M1, K), jnp.float32),
            pltpu.VMEM((CB * M2, K), jnp.float32),
            pltpu.VMEM((CB * M2, K), jnp.float32),
            pltpu.VMEM((CB * K,), jnp.float32),
            pltpu.VMEM((batch // NW,), jnp.float32),
            pltpu.SemaphoreType.DMA,
            pltpu.SemaphoreType.DMA,
            pltpu.SemaphoreType.DMA,
            pltpu.SemaphoreType.DMA,
        ],
        compiler_params=pltpu.CompilerParams(
            needs_layout_passes=False, use_tc_tiling_on_sc=False),
    )
    return run(idx1.reshape(-1), idx2.reshape(-1), V1, V2)


# speedup vs baseline: 1.6604x; 1.1209x over previous
import functools

import jax
import jax.numpy as jnp
from jax import lax
from jax.experimental import pallas as pl
from jax.experimental.pallas import tpu as pltpu
from jax.experimental.pallas import tpu_sc as plsc

K = 16
M1, M2 = 20, 10
NC, NS = 2, 16
NW = NC * NS
CB = 8


def _fm_body(idx1_hbm, idx2_hbm, v1_hbm, v2_hbm, out_hbm,
             idx1_w, idx2_w,
             rows1_a, rows1_b, rows2_a, rows2_b,
             out_v,
             sem1_a, sem1_b, sem2_a, sem2_b,
             *, batch):
    per_w = batch // NW
    nblk = per_w // CB
    wid = lax.axis_index("s") * NC + lax.axis_index("c")
    base = wid * per_w
    rows1_v = (rows1_a, rows1_b)
    rows2_v = (rows2_a, rows2_b)
    sem1 = (sem1_a, sem1_b)
    sem2 = (sem2_a, sem2_b)

    lanes = lax.iota(jnp.int32, K)
    zeros = jnp.zeros((K,), jnp.float32)
    lane0 = lanes == 0

    pltpu.sync_copy(idx1_hbm.at[pl.ds(base * M1, per_w * M1)], idx1_w)
    pltpu.sync_copy(idx2_hbm.at[pl.ds(base * M2, per_w * M2)], idx2_w)

    def fire_rows(tab_hbm, idx_w, rows_v, sem, t, m):
        def grp(g, _):
            iv = idx_w[pl.ds(t * CB * m + g * K, K)]
            for l in range(K):
                r = iv[l]
                pltpu.make_async_copy(
                    tab_hbm.at[pl.ds(r, 1)],
                    rows_v.at[pl.ds(g * K + l, 1)], sem).start()
            return _
        lax.fori_loop(0, CB * m // K, grp, None)

    def start_gathers(t, ph):
        fire_rows(v1_hbm, idx1_w, rows1_v[ph], sem1[ph], t, M1)
        fire_rows(v2_hbm, idx2_w, rows2_v[ph], sem2[ph], t, M2)

    def wait_gathers(ph):
        def w1(n, _):
            pltpu.make_async_copy(v1_hbm.at[pl.ds(0, 1)],
                                  rows1_v[ph].at[pl.ds(0, 1)],
                                  sem1[ph]).wait()
            return _
        def w2(n, _):
            pltpu.make_async_copy(v2_hbm.at[pl.ds(0, 1)],
                                  rows2_v[ph].at[pl.ds(0, 1)],
                                  sem2[ph]).wait()
            return _
        lax.fori_loop(0, CB * M1, w1, None, unroll=8)
        lax.fori_loop(0, CB * M2, w2, None, unroll=8)

    def zero_pass(idx_w, rows_v, t, m):
        def grp(g, _):
            ivec = idx_w[pl.ds(t * CB * m + g * K, K)]
            mz = ivec == 0
            rowids = g * K + lanes
            for k in range(K):
                plsc.store_scatter(
                    rows_v, [rowids, jnp.full((K,), k, jnp.int32)],
                    zeros, mask=mz)
            return _
        lax.fori_loop(0, CB * m // K, grp, None)

    def compute(t, ph):
        zero_pass(idx1_w, rows1_v[ph], t, M1)
        zero_pass(idx2_w, rows2_v[ph], t, M2)
        r1 = rows1_v[ph]
        r2 = rows2_v[ph]

        def elem(i, _):
            s = jnp.zeros((K,), jnp.float32)
            q = jnp.zeros((K,), jnp.float32)
            for j in range(M1):
                r = r1[i * M1 + j]
                s = s + r
                q = q + r * r
            for j in range(M2):
                r = r2[i * M2 + j]
                s = s + r
                q = q + r * r
            red = 0.5 * jnp.sum(s * s - q)
            plsc.store_scatter(out_v, [jnp.full((K,), t * CB + i, jnp.int32)],
                               jnp.broadcast_to(red, (K,)), mask=lane0)
            return _

        lax.fori_loop(0, CB, elem, None)

    start_gathers(0, 0)

    def pair(p, _):
        for ph in range(2):
            t = 2 * p + ph

            @pl.when(t < nblk)
            def _():
                wait_gathers(ph)

                @pl.when(t + 1 < nblk)
                def _():
                    start_gathers(t + 1, 1 - ph)

                compute(t, ph)
        return _

    lax.fori_loop(0, (nblk + 1) // 2, pair, None)
    pltpu.sync_copy(out_v, out_hbm.at[pl.ds(base, per_w)])


def kernel(idx1, idx2, V1, V2):
    batch = idx1.shape[0]
    per_w = batch // NW
    mesh = plsc.VectorSubcoreMesh(
        core_axis_name="c", subcore_axis_name="s",
        num_cores=NC, num_subcores=NS)
    run = pl.kernel(
        functools.partial(_fm_body, batch=batch),
        out_type=jax.ShapeDtypeStruct((batch,), jnp.float32),
        mesh=mesh,
        scratch_types=[
            pltpu.VMEM((per_w * M1,), jnp.int32),
            pltpu.VMEM((per_w * M2,), jnp.int32),
            pltpu.VMEM((CB * M1, K), jnp.float32),
            pltpu.VMEM((CB * M1, K), jnp.float32),
            pltpu.VMEM((CB * M2, K), jnp.float32),
            pltpu.VMEM((CB * M2, K), jnp.float32),
            pltpu.VMEM((per_w,), jnp.float32),
            pltpu.SemaphoreType.DMA,
            pltpu.SemaphoreType.DMA,
            pltpu.SemaphoreType.DMA,
            pltpu.SemaphoreType.DMA,
        ],
        compiler_params=pltpu.CompilerParams(
            needs_layout_passes=False, use_tc_tiling_on_sc=True),
    )
    return run(idx1.reshape(-1), idx2.reshape(-1), V1, V2)


# speedup vs baseline: 1.7162x; 1.0336x over previous
import functools

import jax
import jax.numpy as jnp
from jax import lax
from jax.experimental import pallas as pl
from jax.experimental.pallas import tpu as pltpu
from jax.experimental.pallas import tpu_sc as plsc

K = 16
M1, M2 = 20, 10
NC, NS = 2, 16
NW = NC * NS
CB = 8


def _fm_body(idx1_hbm, idx2_hbm, v1_hbm, v2_hbm, dum1_hbm, dum2_hbm, out_hbm,
             idx1_w, idx2_w,
             rows1_a, rows1_b, rows2_a, rows2_b,
             out_v,
             sem1_a, sem1_b, sem2_a, sem2_b,
             *, batch):
    per_w = batch // NW
    nblk = per_w // CB
    wid = lax.axis_index("s") * NC + lax.axis_index("c")
    base = wid * per_w
    rows1_v = (rows1_a, rows1_b)
    rows2_v = (rows2_a, rows2_b)
    sem1 = (sem1_a, sem1_b)
    sem2 = (sem2_a, sem2_b)

    lanes = lax.iota(jnp.int32, K)
    zeros = jnp.zeros((K,), jnp.float32)
    lane0 = lanes == 0

    pltpu.sync_copy(idx1_hbm.at[pl.ds(base * M1, per_w * M1)], idx1_w)
    pltpu.sync_copy(idx2_hbm.at[pl.ds(base * M2, per_w * M2)], idx2_w)

    def fire_rows(tab_hbm, idx_w, rows_v, sem, t, m):
        def grp(g, _):
            iv = idx_w[pl.ds(t * CB * m + g * K, K)]
            for l in range(K):
                r = iv[l]
                pltpu.make_async_copy(
                    tab_hbm.at[pl.ds(r, 1)],
                    rows_v.at[pl.ds(g * K + l, 1)], sem).start()
            return _
        lax.fori_loop(0, CB * m // K, grp, None)

    def start_gathers(t, ph):
        fire_rows(v1_hbm, idx1_w, rows1_v[ph], sem1[ph], t, M1)
        fire_rows(v2_hbm, idx2_w, rows2_v[ph], sem2[ph], t, M2)

    def wait_gathers(ph):
        pltpu.make_async_copy(dum1_hbm, rows1_v[ph], sem1[ph]).wait()
        pltpu.make_async_copy(dum2_hbm, rows2_v[ph], sem2[ph]).wait()

    def zero_pass(idx_w, rows_v, t, m):
        def grp(g, _):
            ivec = idx_w[pl.ds(t * CB * m + g * K, K)]
            mz = ivec == 0
            rowids = g * K + lanes
            for k in range(K):
                plsc.store_scatter(
                    rows_v, [rowids, jnp.full((K,), k, jnp.int32)],
                    zeros, mask=mz)
            return _
        lax.fori_loop(0, CB * m // K, grp, None)

    def compute(t, ph):
        zero_pass(idx1_w, rows1_v[ph], t, M1)
        zero_pass(idx2_w, rows2_v[ph], t, M2)
        r1 = rows1_v[ph]
        r2 = rows2_v[ph]

        def elem(i, _):
            s = jnp.zeros((K,), jnp.float32)
            q = jnp.zeros((K,), jnp.float32)
            for j in range(M1):
                r = r1[i * M1 + j]
                s = s + r
                q = q + r * r
            for j in range(M2):
                r = r2[i * M2 + j]
                s = s + r
                q = q + r * r
            red = 0.5 * jnp.sum(s * s - q)
            plsc.store_scatter(out_v, [jnp.full((K,), t * CB + i, jnp.int32)],
                               jnp.broadcast_to(red, (K,)), mask=lane0)
            return _

        lax.fori_loop(0, CB, elem, None)

    start_gathers(0, 0)

    def pair(p, _):
        for ph in range(2):
            t = 2 * p + ph

            @pl.when(t < nblk)
            def _():
                wait_gathers(ph)

                @pl.when(t + 1 < nblk)
                def _():
                    start_gathers(t + 1, 1 - ph)

                compute(t, ph)
        return _

    lax.fori_loop(0, (nblk + 1) // 2, pair, None)
    pltpu.sync_copy(out_v, out_hbm.at[pl.ds(base, per_w)])


def kernel(idx1, idx2, V1, V2):
    batch = idx1.shape[0]
    per_w = batch // NW
    mesh = plsc.VectorSubcoreMesh(
        core_axis_name="c", subcore_axis_name="s",
        num_cores=NC, num_subcores=NS)
    run = pl.kernel(
        functools.partial(_fm_body, batch=batch),
        out_type=jax.ShapeDtypeStruct((batch,), jnp.float32),
        mesh=mesh,
        scratch_types=[
            pltpu.VMEM((per_w * M1,), jnp.int32),
            pltpu.VMEM((per_w * M2,), jnp.int32),
            pltpu.VMEM((CB * M1, K), jnp.float32),
            pltpu.VMEM((CB * M1, K), jnp.float32),
            pltpu.VMEM((CB * M2, K), jnp.float32),
            pltpu.VMEM((CB * M2, K), jnp.float32),
            pltpu.VMEM((per_w,), jnp.float32),
            pltpu.SemaphoreType.DMA,
            pltpu.SemaphoreType.DMA,
            pltpu.SemaphoreType.DMA,
            pltpu.SemaphoreType.DMA,
        ],
        compiler_params=pltpu.CompilerParams(
            needs_layout_passes=False, use_tc_tiling_on_sc=True),
    )
    return run(idx1.reshape(-1), idx2.reshape(-1), V1, V2,
               jnp.zeros((CB * M1, K), jnp.float32),
               jnp.zeros((CB * M2, K), jnp.float32))
